# baseline (speedup 1.0000x reference)
import functools

import jax
import jax.numpy as jnp
import numpy as np
from jax import lax
from jax.experimental import pallas as pl
from jax.experimental.pallas import tpu as pltpu
from jax.experimental.pallas import tpu_sc as plsc

NC = 2
NS = 16
NW = NC * NS

BBLK = 128
L = 16


def _positional_encoding(model_size, sequence_length):
    pos = np.arange(sequence_length, dtype=np.float64)[:, None]
    i = np.arange(model_size, dtype=np.float64)[None, :]
    exponent = np.where(i % 2 == 0, i, i - 1) / model_size
    angle = pos / np.power(10000.0, exponent)
    pe = np.where(i % 2 == 0, np.sin(angle), np.cos(angle))
    return pe.astype(np.float32)


@functools.partial(jax.jit, static_argnums=())
def _sc_embed(idx_t, pe_b, table):
    seq, batch = idx_t.shape
    dim = pe_b.shape[1] // L
    dpad = table.shape[1]

    mesh = plsc.VectorSubcoreMesh(
        core_axis_name="c", subcore_axis_name="s",
        num_cores=NC, num_subcores=NS)

    @functools.partial(
        pl.kernel,
        mesh=mesh,
        compiler_params=pltpu.CompilerParams(needs_layout_passes=False),
        out_type=jax.ShapeDtypeStruct((seq, dim, batch), jnp.float32),
        scratch_types=[
            pltpu.VMEM((seq, BBLK), jnp.int32),
            pltpu.VMEM((2, BBLK, dpad), jnp.float32),
            pltpu.VMEM((2, dim, BBLK), jnp.float32),
            pltpu.VMEM((2, dim * L), jnp.float32),
            pltpu.SemaphoreType.DMA((2,)),
            pltpu.SemaphoreType.DMA((2,)),
            pltpu.SemaphoreType.DMA((2,)),
        ],
    )
    def k(idx_hbm, pe_hbm, table_hbm, out_hbm,
          idx_v, rows_v, tile_v, pe_v, gsem, psem, osem):
        wid = lax.axis_index("s") * NC + lax.axis_index("c")
        b0 = wid * BBLK

        pltpu.sync_copy(idx_hbm.at[:, pl.ds(b0, BBLK)], idx_v)

        iota = lax.iota(jnp.int32, L)

        def step(s, _):
            p = s % 2

            @pl.when(s < seq)
            def _():
                pltpu.async_copy(
                    table_hbm.at[idx_v.at[s]], rows_v.at[p], gsem.at[p])
                pltpu.async_copy(pe_hbm.at[s], pe_v.at[p], psem.at[p])

            @pl.when(s >= 1)
            def _():
                j = s - 1
                jp = j % 2
                pltpu.make_async_copy(
                    table_hbm.at[idx_v.at[j]], rows_v.at[jp],
                    gsem.at[jp]).wait()
                pltpu.make_async_copy(
                    pe_hbm.at[j], pe_v.at[jp], psem.at[jp]).wait()

                @pl.when(s >= 3)
                def _():
                    pltpu.make_async_copy(
                        tile_v.at[jp],
                        out_hbm.at[0].at[:, pl.ds(0, BBLK)],
                        osem.at[jp]).wait()

                src = rows_v.at[jp]
                dst = tile_v.at[jp]
                for d in range(dim):
                    pev = pe_v[jp, pl.ds(d * L, L)]
                    for g in range(BBLK // L):
                        ivec = iota + (g * L)
                        dvec = jnp.full((L,), d, jnp.int32)
                        got = plsc.load_gather(src, [ivec, dvec])
                        dst[d, pl.ds(g * L, L)] = got + pev
                pltpu.async_copy(
                    dst, out_hbm.at[j].at[:, pl.ds(b0, BBLK)], osem.at[jp])
            return ()

        lax.fori_loop(0, seq + 1, step, ())
        for jp in range(2):
            pltpu.make_async_copy(
                tile_v.at[jp], out_hbm.at[0].at[:, pl.ds(0, BBLK)],
                osem.at[jp]).wait()

    return k(idx_t, pe_b, table)


def kernel(inputs, table):
    batch, seq = inputs.shape
    vocab, dim = table.shape
    dpad = 128
    assert batch % (NW * BBLK) == 0 or batch == NW * BBLK

    table128 = jnp.pad(table, ((0, 0), (0, dpad - dim)))
    idx_t = inputs.astype(jnp.int32).T
    pe = _positional_encoding(dim, seq)
    pe_b = jnp.asarray(
        np.repeat(pe[:, :, None], L, axis=2).reshape(seq, dim * L))
    out3 = _sc_embed(idx_t, pe_b, table128)
    return out3.transpose(2, 0, 1)


# speedup vs baseline: 1.8968x; 1.8968x over previous
import functools

import jax
import jax.numpy as jnp
import numpy as np
from jax import lax
from jax.experimental import pallas as pl
from jax.experimental.pallas import tpu as pltpu
from jax.experimental.pallas import tpu_sc as plsc

NC = 2
NS = 16
NW = NC * NS

STREAM = 128
NBUF = 4


def _positional_encoding(model_size, sequence_length):
    pos = np.arange(sequence_length, dtype=np.float64)[:, None]
    i = np.arange(model_size, dtype=np.float64)[None, :]
    exponent = np.where(i % 2 == 0, i, i - 1) / model_size
    angle = pos / np.power(10000.0, exponent)
    pe = np.where(i % 2 == 0, np.sin(angle), np.cos(angle))
    return pe.astype(np.float32)


def _pe_phases(dim, seq, dim_pad):
    n_phases = seq // np.gcd(STREAM, seq)
    pe = _positional_encoding(dim, seq)
    out = np.zeros((n_phases, STREAM, dim_pad), dtype=np.float32)
    for k in range(n_phases):
        o = (STREAM * k) % seq
        rows = (o + np.arange(STREAM)) % seq
        out[k, :, :dim] = pe[rows]
    return out


@functools.partial(jax.jit, static_argnums=(3,))
def _sc_embed(idx, pe_all, table, n_streams):
    n_phases, _, dpad = pe_all.shape
    b_per_w = n_streams * STREAM
    B = NW * b_per_w

    mesh = plsc.VectorSubcoreMesh(
        core_axis_name="c", subcore_axis_name="s",
        num_cores=NC, num_subcores=NS)

    @functools.partial(
        pl.kernel,
        mesh=mesh,
        out_type=jax.ShapeDtypeStruct((B, dpad), jnp.float32),
        scratch_types=[
            pltpu.VMEM((n_streams, STREAM), jnp.int32),
            pltpu.VMEM((NBUF, STREAM, dpad), jnp.float32),
            pltpu.VMEM_SHARED((n_phases, STREAM, dpad), jnp.float32),
            pltpu.SemaphoreType.DMA((NBUF,)),
            pltpu.SemaphoreType.DMA((NBUF,)),
            pltpu.SemaphoreType.DMA((NBUF,)),
        ],
    )
    def k(idx_hbm, pe_hbm, table_hbm, out_hbm,
          idx_v, rows_v, pe_sh, psem, gsem, osem):
        tview = table_hbm
        sid = lax.axis_index("s")
        wid = sid * NC + lax.axis_index("c")
        base = wid * b_per_w

        @pl.when(sid == 0)
        def _():
            pltpu.sync_copy(pe_hbm, pe_sh)
        pltpu.sync_copy(idx_hbm.at[wid], idx_v)
        plsc.subcore_barrier()

        def mi_body(mi, _):
            for b in range(NBUF):
                j = mi * NBUF + b

                @pl.when(mi >= 1)
                def _(b=b):
                    pltpu.make_async_copy(
                        rows_v.at[b], out_hbm.at[pl.ds(0, STREAM)],
                        osem.at[b]).wait()
                pltpu.async_copy(
                    pe_sh.at[lax.rem(j, n_phases)], rows_v.at[b], psem.at[b])

            for b in range(NBUF):
                j = mi * NBUF + b
                pltpu.make_async_copy(
                    pe_sh.at[0], rows_v.at[b], psem.at[b]).wait()
                pltpu.async_copy(
                    tview.at[idx_v.at[j]], rows_v.at[b],
                    gsem.at[b], add=True)

            for b in range(NBUF):
                j = mi * NBUF + b
                pltpu.make_async_copy(
                    tview.at[idx_v.at[j]], rows_v.at[b], gsem.at[b]).wait()
                pltpu.async_copy(
                    rows_v.at[b],
                    out_hbm.at[pl.ds(base + j * STREAM, STREAM)], osem.at[b])
            return ()

        lax.fori_loop(0, n_streams // NBUF, mi_body, ())

        for b in range(NBUF):
            pltpu.make_async_copy(
                rows_v.at[b], out_hbm.at[pl.ds(0, STREAM)], osem.at[b]).wait()

    return k(idx, pe_all, table)


def kernel(inputs, table):
    batch, seq = inputs.shape
    vocab, dim = table.shape
    B = batch * seq
    dpad = 128
    assert B % (NW * STREAM) == 0
    n_streams = B // (NW * STREAM)
    assert (n_streams * STREAM) % seq == 0

    table128 = jnp.pad(table.T, ((0, dpad - dim), (0, 0))).T
    idx = inputs.astype(jnp.int32).reshape(NW, n_streams, STREAM)
    pe_all = jnp.asarray(_pe_phases(dim, seq, dpad))
    out = _sc_embed(idx, pe_all, table128, n_streams)
    return out.reshape(batch, seq, dpad)[:, :, :dim]
